# baseline (speedup 1.0000x reference)
import functools

import jax
import jax.numpy as jnp
from jax import lax
from jax.experimental import pallas as pl
from jax.experimental.pallas import tpu as pltpu

GRID_X = 256
GRID_Y = 256
NCELL = GRID_X * GRID_Y
MAX_VOXELS = 12000
MAX_PTS = 32
VB = 512
BVP = 24576


def _voxelize(points):
    B, N, _ = points.shape
    sent = NCELL

    def one(pts):
        m = jnp.all((pts[:, :3] >= 0.0) & (pts[:, :3] < 1.0), axis=1)
        vx = jnp.clip((pts[:, 0] * GRID_X).astype(jnp.int32), 0, GRID_X - 1)
        vy = jnp.clip((pts[:, 1] * GRID_Y).astype(jnp.int32), 0, GRID_Y - 1)
        h = vx * GRID_Y + vy
        hs = jnp.where(m, h, sent)
        pos = jnp.arange(N, dtype=jnp.int32)
        sh, order = lax.sort((hs, pos), num_keys=1, is_stable=True)
        is_new = jnp.concatenate([jnp.array([True]), sh[1:] != sh[:-1]])
        sinv = jnp.cumsum(is_new.astype(jnp.int32)) - 1
        first = lax.cummax(jnp.where(is_new, pos, -1))
        within = pos - first
        keep = (sh < sent) & (within < MAX_PTS) & (sinv < MAX_VOXELS)
        si = jnp.where(keep, sinv, MAX_VOXELS)
        wi = jnp.where(keep, within, MAX_PTS)
        psorted = pts[order]
        vox = jnp.zeros((MAX_VOXELS, MAX_PTS, 4), jnp.float32).at[si, wi].set(
            psorted, mode='drop')
        val = jnp.zeros((MAX_VOXELS, MAX_PTS), jnp.float32).at[si, wi].set(
            1.0, mode='drop')
        newu = is_new & (sh < sent) & (sinv < MAX_VOXELS)
        ui = jnp.where(newu, sinv, MAX_VOXELS)
        uh = jnp.zeros((MAX_VOXELS,), jnp.int32).at[ui].set(
            sh.astype(jnp.int32), mode='drop')
        return vox, val, uh

    return jax.vmap(one)(points)


def _pass1_body(vox_ref, cen_ref, w1ct_ref, w1bt_ref, stat_ref, acc_ref):
    cterm = jnp.dot(w1bt_ref[...], cen_ref[:3, :],
                    preferred_element_type=jnp.float32)
    zs = jnp.zeros((64, VB), jnp.float32)
    zs2 = jnp.zeros((64, VB), jnp.float32)
    for k in range(MAX_PTS):
        z = jnp.dot(w1ct_ref[...], vox_ref[k],
                    preferred_element_type=jnp.float32) - cterm
        zs += z
        zs2 += z * z

    @pl.when(pl.program_id(0) == 0)
    def _():
        acc_ref[...] = jnp.zeros_like(acc_ref)

    acc_ref[0, :] += jnp.sum(zs, axis=1)
    acc_ref[1, :] += jnp.sum(zs2, axis=1)

    @pl.when(pl.program_id(0) == pl.num_programs(0) - 1)
    def _():
        stat_ref[...] = acc_ref[...]


def _pass2_body(vox_ref, cen_ref, w1ct_ref, w1bt_ref, a1_ref, c1_ref,
                w2t_ref, stat_ref, acc_ref):
    cterm = jnp.dot(w1bt_ref[...], cen_ref[:3, :],
                    preferred_element_type=jnp.float32)
    zs = jnp.zeros((64, VB), jnp.float32)
    zs2 = jnp.zeros((64, VB), jnp.float32)
    for k in range(MAX_PTS):
        z = jnp.dot(w1ct_ref[...], vox_ref[k],
                    preferred_element_type=jnp.float32) - cterm
        h1 = jnp.maximum(z * a1_ref[...] + c1_ref[...], 0.0)
        z2 = jnp.dot(w2t_ref[...], h1, preferred_element_type=jnp.float32)
        zs += z2
        zs2 += z2 * z2

    @pl.when(pl.program_id(0) == 0)
    def _():
        acc_ref[...] = jnp.zeros_like(acc_ref)

    acc_ref[0, :] += jnp.sum(zs, axis=1)
    acc_ref[1, :] += jnp.sum(zs2, axis=1)

    @pl.when(pl.program_id(0) == pl.num_programs(0) - 1)
    def _():
        stat_ref[...] = acc_ref[...]


def _pass3_body(vox_ref, val_ref, cen_ref, w1ct_ref, w1bt_ref, a1_ref,
                c1_ref, w2t_ref, a2_ref, c2_ref, out_ref):
    cterm = jnp.dot(w1bt_ref[...], cen_ref[:3, :],
                    preferred_element_type=jnp.float32)
    m = jnp.zeros((64, VB), jnp.float32)
    for k in range(MAX_PTS):
        z = jnp.dot(w1ct_ref[...], vox_ref[k],
                    preferred_element_type=jnp.float32) - cterm
        h1 = jnp.maximum(z * a1_ref[...] + c1_ref[...], 0.0)
        z2 = jnp.dot(w2t_ref[...], h1, preferred_element_type=jnp.float32)
        h2 = jnp.maximum(z2 * a2_ref[...] + c2_ref[...], 0.0) * val_ref[k]
        m = jnp.maximum(m, h2)
    out_ref[...] = m


def _mlp_pipeline(vox_s, val_s, cen, W1, g1, be1, W2, g2, be2, n_rows):
    nblk = BVP // VB
    Ntot = jnp.float32(n_rows)
    npad = jnp.float32(BVP * MAX_PTS - n_rows)
    w1b = W1[4:7, :] + W1[7:10, :]
    w1c = W1[0:4, :].at[0:3, :].add(w1b)
    w1ct = jnp.transpose(w1c)
    w1bt = jnp.transpose(w1b)
    w2t = jnp.transpose(W2)

    grid = (nblk,)
    vox_spec = pl.BlockSpec((MAX_PTS, 4, VB), lambda i: (0, 0, i))
    val_spec = pl.BlockSpec((MAX_PTS, 1, VB), lambda i: (0, 0, i))
    cen_spec = pl.BlockSpec((4, VB), lambda i: (0, i))
    w1ct_spec = pl.BlockSpec((64, 4), lambda i: (0, 0))
    w1bt_spec = pl.BlockSpec((64, 3), lambda i: (0, 0))
    vec_spec = pl.BlockSpec((64, 1), lambda i: (0, 0))
    w2_spec = pl.BlockSpec((64, 64), lambda i: (0, 0))
    stat_spec = pl.BlockSpec((2, 64), lambda i: (0, 0))

    stats1 = pl.pallas_call(
        _pass1_body,
        grid=grid,
        in_specs=[vox_spec, cen_spec, w1ct_spec, w1bt_spec],
        out_specs=stat_spec,
        out_shape=jax.ShapeDtypeStruct((2, 64), jnp.float32),
        scratch_shapes=[pltpu.VMEM((2, 64), jnp.float32)],
    )(vox_s, cen, w1ct, w1bt)

    mz1 = stats1[0] / Ntot
    var1 = stats1[1] / Ntot - mz1 * mz1
    sc1 = g1 / jnp.sqrt(var1 + 1e-5)
    a1 = sc1[:, None]
    c1 = (be1 - mz1 * sc1)[:, None]

    stats2 = pl.pallas_call(
        _pass2_body,
        grid=grid,
        in_specs=[vox_spec, cen_spec, w1ct_spec, w1bt_spec, vec_spec,
                  vec_spec, w2_spec],
        out_specs=stat_spec,
        out_shape=jax.ShapeDtypeStruct((2, 64), jnp.float32),
        scratch_shapes=[pltpu.VMEM((2, 64), jnp.float32)],
    )(vox_s, cen, w1ct, w1bt, a1, c1, w2t)

    z2p = jnp.dot(jnp.maximum(c1[:, 0], 0.0), W2)
    mz2 = (stats2[0] - npad * z2p) / Ntot
    var2 = (stats2[1] - npad * z2p * z2p) / Ntot - mz2 * mz2
    sc2 = g2 / jnp.sqrt(var2 + 1e-5)
    a2 = sc2[:, None]
    c2 = (be2 - mz2 * sc2)[:, None]

    feats_t = pl.pallas_call(
        _pass3_body,
        grid=grid,
        in_specs=[vox_spec, val_spec, cen_spec, w1ct_spec, w1bt_spec,
                  vec_spec, vec_spec, w2_spec, vec_spec, vec_spec],
        out_specs=pl.BlockSpec((64, VB), lambda i: (0, i)),
        out_shape=jax.ShapeDtypeStruct((64, BVP), jnp.float32),
    )(vox_s, val_s, cen, w1ct, w1bt, a1, c1, w2t, a2, c2)
    return feats_t


def kernel(points, W1, b1, g1, be1, W2, b2, g2, be2):
    B = points.shape[0]
    BV = B * MAX_VOXELS
    del b1, b2

    vox, val, uh = _voxelize(points)

    psum = jnp.sum(vox, axis=2)
    pcnt = jnp.maximum(jnp.sum(val, axis=2), 1.0)
    cen = psum / pcnt[..., None]

    pad = BVP - BV
    vox_s = jnp.pad(
        jnp.transpose(vox.reshape(BV, MAX_PTS, 4), (1, 2, 0)),
        ((0, 0), (0, 0), (0, pad)))
    val_s = jnp.pad(
        jnp.transpose(val.reshape(BV, MAX_PTS), (1, 0))[:, None, :],
        ((0, 0), (0, 0), (0, pad)))
    cen_t = jnp.pad(jnp.transpose(cen.reshape(BV, 4)),
                    ((0, 0), (0, pad)))

    feats_t = _mlp_pipeline(vox_s, val_s, cen_t, W1, g1, be1, W2, g2, be2,
                            BV * MAX_PTS)
    feats = jnp.transpose(feats_t[:, :BV]).reshape(B, MAX_VOXELS, 64)

    x = uh // GRID_Y
    y = uh % GRID_Y
    bev = jnp.zeros((B, GRID_Y, GRID_X, 64), jnp.float32)
    bev = bev.at[jnp.arange(B)[:, None], y, x].set(feats)
    return jnp.transpose(bev, (0, 3, 1, 2))
